# baseline (speedup 1.0000x reference)
import functools

import jax
import jax.numpy as jnp
from jax import lax
from jax.experimental import pallas as pl
from jax.experimental.pallas import tpu as pltpu
from jax.experimental.pallas import tpu_sc as plsc

_N = 10000
_E = 320000
_D = 128
_HD = _D // 2
_NC = 2
_NS = 16
_CH = 64
_EPT = 20480
_EPAD = _NS * _EPT
_NCHUNK = _EPT // _CH
_PIECE = 32
_NPIECE = _NCHUNK // _PIECE
_NBUF = 4
_LOOKAHEAD = _NBUF - 1
_WB = 40
_NRB = _N // _WB
_NRB_CEIL = -(-_NRB // _NS)
_FSR = _N // _NS
_VL = 16


def _sc_body(feat_hbm, rows_hbm, cols_hbm, vals_hbm, part_hbm,
             acc, feat_sp, rowsb, colsb, valsb,
             gbuf0, gbuf1, gbuf2, gbuf3,
             gsem0, gsem1, gsem2, gsem3,
             ssem0, ssem1, ssem2, ssem3):
    gbufs = [gbuf0, gbuf1, gbuf2, gbuf3]
    gsems = [gsem0, gsem1, gsem2, gsem3]
    ssems = [ssem0, ssem1, ssem2, ssem3]
    c = lax.axis_index("c")
    s = lax.axis_index("s")

    pltpu.sync_copy(feat_hbm.at[c, pl.ds(s * _FSR, _FSR)],
                    feat_sp.at[pl.ds(s * _FSR, _FSR)])

    @pl.loop(0, _WB)
    def _zero(i):
        for j in range(_HD // _VL):
            gbuf0[i, pl.ds(j * _VL, _VL)] = jnp.zeros((_VL,), jnp.float32)

    for bi in range(_NRB_CEIL):
        b = bi * _NS + s

        @pl.when(b < _NRB)
        def _():
            pltpu.sync_copy(gbuf0.at[pl.ds(0, _WB)],
                            acc.at[pl.ds(b * _WB, _WB)])

    plsc.subcore_barrier()

    @pl.loop(0, _NPIECE)
    def _piece(p):
        sl_p = pl.ds(p * _PIECE, _PIECE)
        pltpu.sync_copy(rows_hbm.at[s, sl_p], rowsb)
        pltpu.sync_copy(cols_hbm.at[s, sl_p], colsb)
        pltpu.sync_copy(vals_hbm.at[s, sl_p], valsb)

        for k0 in range(_LOOKAHEAD):
            pltpu.async_copy(feat_sp.at[colsb.at[k0]], gbufs[k0],
                             gsems[k0])

        @pl.loop(0, _PIECE, step=_NBUF)
        def _wave(kbase):
            for j in range(_NBUF):
                k = kbase + j
                gb, gs, ss = gbufs[j], gsems[j], ssems[j]
                pltpu.make_async_copy(feat_sp.at[colsb.at[k]], gb,
                                      gs).wait()

                @pl.loop(0, _CH // _VL)
                def _scale(g):
                    vv = valsb[k, pl.ds(g * _VL, _VL)]
                    for l in range(_VL):
                        e = g * _VL + l
                        v = vv[l]
                        for jj in range(_HD // _VL):
                            sl = pl.ds(jj * _VL, _VL)
                            gb[e, sl] = gb[e, sl] * v

                pltpu.async_copy(gb, acc.at[pl.ds(0, _CH)], ss)

                jd = (j + _LOOKAHEAD) % _NBUF
                kd = k - 1

                @pl.when(kd >= 0)
                def _():
                    pltpu.make_async_copy(gbufs[jd],
                                          acc.at[pl.ds(0, _CH)],
                                          ssems[jd]).wait()

                @pl.when(k + _LOOKAHEAD < _PIECE)
                def _():
                    pltpu.async_copy(
                        feat_sp.at[colsb.at[k + _LOOKAHEAD]],
                        gbufs[jd], gsems[jd])

        pltpu.make_async_copy(gbufs[(_PIECE - 1) % _NBUF],
                              acc.at[pl.ds(0, _CH)],
                              ssems[(_PIECE - 1) % _NBUF]).wait()

    plsc.subcore_barrier()

    for bi in range(_NRB_CEIL):
        b = bi * _NS + s

        @pl.when(b < _NRB)
        def _():
            pltpu.sync_copy(acc.at[pl.ds(b * _WB, _WB)],
                            gbuf0.at[pl.ds(0, _WB)])
            pltpu.sync_copy(gbuf0.at[pl.ds(0, _WB)],
                            part_hbm.at[c, pl.ds(b * _WB, _WB)])


@functools.lru_cache(maxsize=1)
def _get_sc_spmm():
    return pl.kernel(
        _sc_body,
        out_type=jax.ShapeDtypeStruct((_NC, _N, _HD), jnp.float32),
        mesh=plsc.VectorSubcoreMesh(core_axis_name="c", subcore_axis_name="s"),
        compiler_params=pltpu.CompilerParams(use_tc_tiling_on_sc=False),
        scratch_types=[
            pltpu.VMEM_SHARED((_N, _HD), jnp.float32),
            pltpu.VMEM_SHARED((_N, _HD), jnp.float32),
            pltpu.VMEM((_PIECE, _CH), jnp.int32),
            pltpu.VMEM((_PIECE, _CH), jnp.int32),
            pltpu.VMEM((_PIECE, _CH), jnp.float32),
        ] + [pltpu.VMEM((_CH, _HD), jnp.float32)] * _NBUF
          + [pltpu.SemaphoreType.DMA] * (2 * _NBUF),
    )


def _norm(h, scale, offset):
    m = jnp.mean(h, axis=1, keepdims=True)
    d = h - m
    v = jnp.mean(d * d, axis=1, keepdims=True) + 1e-9
    return d * scale * lax.rsqrt(v) + offset


def _dense_body(x_ref, p_ref, w0t_ref, w1t_ref, b0_ref, b1_ref,
                s0_ref, o0_ref, s1_ref, o1_ref, out_ref):
    x = x_ref[...]
    h0 = jnp.maximum(
        jnp.dot(x, w0t_ref[...], preferred_element_type=jnp.float32)
        + b0_ref[...], 0.0)
    n0 = _norm(h0, s0_ref[...], o0_ref[...])
    hop1 = jnp.concatenate([p_ref[0], p_ref[1]], axis=1)
    h1 = jnp.maximum(
        jnp.dot(hop1, w1t_ref[...], preferred_element_type=jnp.float32)
        + b1_ref[...], 0.0)
    n1 = _norm(h1, s1_ref[...], o1_ref[...])
    out_ref[...] = jnp.concatenate([n0, n1], axis=1)


_RB = 400

_dense = pl.pallas_call(
    _dense_body,
    grid=(_N // _RB,),
    in_specs=[
        pl.BlockSpec((_RB, _D), lambda i: (i, 0)),
        pl.BlockSpec((_NC, _RB, _HD), lambda i: (0, i, 0)),
        pl.BlockSpec((_D, _D), lambda i: (0, 0)),
        pl.BlockSpec((_D, _D), lambda i: (0, 0)),
        pl.BlockSpec((1, _D), lambda i: (0, 0)),
        pl.BlockSpec((1, _D), lambda i: (0, 0)),
        pl.BlockSpec((1, _D), lambda i: (0, 0)),
        pl.BlockSpec((1, _D), lambda i: (0, 0)),
        pl.BlockSpec((1, _D), lambda i: (0, 0)),
        pl.BlockSpec((1, _D), lambda i: (0, 0)),
    ],
    out_specs=pl.BlockSpec((_RB, 2 * _D), lambda i: (i, 0)),
    out_shape=jax.ShapeDtypeStruct((_N, 2 * _D), jnp.float32),
)


def kernel(feat_in, edge_index, edge_values, W0, W1, b0, b1,
           offset0, offset1, scale0, scale1):
    pad = _EPAD - _E
    rows3 = jnp.concatenate(
        [edge_index[0], jnp.zeros((pad,), jnp.int32)]).reshape(
            _NS, _NCHUNK, _CH)
    cols3 = jnp.concatenate(
        [edge_index[1], jnp.zeros((pad,), jnp.int32)]).reshape(
            _NS, _NCHUNK, _CH)
    vals3 = jnp.concatenate(
        [edge_values, jnp.zeros((pad,), jnp.float32)]).reshape(
            _NS, _NCHUNK, _CH)
    feat_halves = jnp.stack([feat_in[:, :_HD], feat_in[:, _HD:]])
    part = _get_sc_spmm()(feat_halves, rows3, cols3, vals3)
    return _dense(
        feat_in, part, W0.T, W1.T,
        b0.reshape(1, _D), b1.reshape(1, _D),
        scale0.reshape(1, _D), offset0.reshape(1, _D),
        scale1.reshape(1, _D), offset1.reshape(1, _D),
    )


# speedup vs baseline: 1.8349x; 1.8349x over previous
import functools

import jax
import jax.numpy as jnp
from jax import lax
from jax.experimental import pallas as pl
from jax.experimental.pallas import tpu as pltpu
from jax.experimental.pallas import tpu_sc as plsc

_N = 10000
_E = 320000
_D = 128
_HD = _D // 2
_NC = 2
_NS = 16
_CH = 64
_EPT = 20480
_EPAD = _NS * _EPT
_NCHUNK = _EPT // _CH
_PIECE = 32
_NPIECE = _NCHUNK // _PIECE
_NBUF = 4
_LOOKAHEAD = _NBUF - 1
_WB = 40
_NRB = _N // _WB
_NRB_CEIL = -(-_NRB // _NS)
_FSR = _N // _NS
_VL = 16


def _sc_body(feat_hbm, rows_hbm, cols_hbm, vals_hbm, part_hbm,
             acc, feat_sp, rowsb, colsb, valsb,
             gbuf0, gbuf1, gbuf2, gbuf3,
             gsem0, gsem1, gsem2, gsem3,
             ssem0, ssem1, ssem2, ssem3):
    gbufs = [gbuf0, gbuf1, gbuf2, gbuf3]
    gsems = [gsem0, gsem1, gsem2, gsem3]
    ssems = [ssem0, ssem1, ssem2, ssem3]
    c = lax.axis_index("c")
    s = lax.axis_index("s")

    pltpu.sync_copy(feat_hbm.at[c, pl.ds(s * _FSR, _FSR)],
                    feat_sp.at[pl.ds(s * _FSR, _FSR)])

    @pl.loop(0, _WB)
    def _zero(i):
        for j in range(_HD // _VL):
            gbuf0[i, pl.ds(j * _VL, _VL)] = jnp.zeros((_VL,), jnp.float32)

    for bi in range(_NRB_CEIL):
        b = bi * _NS + s

        @pl.when(b < _NRB)
        def _():
            pltpu.sync_copy(gbuf0.at[pl.ds(0, _WB)],
                            acc.at[pl.ds(b * _WB, _WB)])

    plsc.subcore_barrier()

    @pl.loop(0, _NPIECE)
    def _piece(p):
        sl_p = pl.ds(p * _PIECE, _PIECE)
        pltpu.sync_copy(rows_hbm.at[s, sl_p], rowsb)
        pltpu.sync_copy(cols_hbm.at[s, sl_p], colsb)
        pltpu.sync_copy(vals_hbm.at[s, sl_p], valsb)

        for k0 in range(_LOOKAHEAD):
            pltpu.async_copy(feat_sp.at[colsb.at[k0]], gbufs[k0],
                             gsems[k0])

        @pl.loop(0, _PIECE, step=_NBUF)
        def _wave(kbase):
            for j in range(_NBUF):
                k = kbase + j
                gb, gs, ss = gbufs[j], gsems[j], ssems[j]
                pltpu.make_async_copy(feat_sp.at[colsb.at[k]], gb,
                                      gs).wait()

                @plsc.parallel_loop(0, _CH // _VL, unroll=2)
                def _scale(g):
                    vv = valsb[k, pl.ds(g * _VL, _VL)]
                    for l in range(_VL):
                        e = g * _VL + l
                        v = vv[l]
                        for jj in range(_HD // _VL):
                            sl = pl.ds(jj * _VL, _VL)
                            gb[e, sl] = gb[e, sl] * v

                pltpu.async_copy(gb, acc.at[rowsb.at[k]], ss, add=True)

                jd = (j + _LOOKAHEAD) % _NBUF
                kd = k - 1

                @pl.when(kd >= 0)
                def _():
                    pltpu.make_async_copy(gbufs[jd],
                                          acc.at[rowsb.at[k]],
                                          ssems[jd]).wait()

                @pl.when(k + _LOOKAHEAD < _PIECE)
                def _():
                    pltpu.async_copy(
                        feat_sp.at[colsb.at[k + _LOOKAHEAD]],
                        gbufs[jd], gsems[jd])

        pltpu.make_async_copy(gbufs[(_PIECE - 1) % _NBUF],
                              acc.at[rowsb.at[_PIECE - 1]],
                              ssems[(_PIECE - 1) % _NBUF]).wait()

    plsc.subcore_barrier()

    for bi in range(_NRB_CEIL):
        b = bi * _NS + s

        @pl.when(b < _NRB)
        def _():
            pltpu.sync_copy(acc.at[pl.ds(b * _WB, _WB)],
                            gbuf0.at[pl.ds(0, _WB)])
            pltpu.sync_copy(gbuf0.at[pl.ds(0, _WB)],
                            part_hbm.at[c, pl.ds(b * _WB, _WB)])


@functools.lru_cache(maxsize=1)
def _get_sc_spmm():
    return pl.kernel(
        _sc_body,
        out_type=jax.ShapeDtypeStruct((_NC, _N, _HD), jnp.float32),
        mesh=plsc.VectorSubcoreMesh(core_axis_name="c", subcore_axis_name="s"),
        compiler_params=pltpu.CompilerParams(use_tc_tiling_on_sc=False),
        scratch_types=[
            pltpu.VMEM_SHARED((_N, _HD), jnp.float32),
            pltpu.VMEM_SHARED((_N, _HD), jnp.float32),
            pltpu.VMEM((_PIECE, _CH), jnp.int32),
            pltpu.VMEM((_PIECE, _CH), jnp.int32),
            pltpu.VMEM((_PIECE, _CH), jnp.float32),
        ] + [pltpu.VMEM((_CH, _HD), jnp.float32)] * _NBUF
          + [pltpu.SemaphoreType.DMA] * (2 * _NBUF),
    )


def _norm(h, scale, offset):
    m = jnp.mean(h, axis=1, keepdims=True)
    d = h - m
    v = jnp.mean(d * d, axis=1, keepdims=True) + 1e-9
    return d * scale * lax.rsqrt(v) + offset


def _dense_body(x_ref, p_ref, w0t_ref, w1t_ref, b0_ref, b1_ref,
                s0_ref, o0_ref, s1_ref, o1_ref, out_ref):
    x = x_ref[...]
    h0 = jnp.maximum(
        jnp.dot(x, w0t_ref[...], preferred_element_type=jnp.float32)
        + b0_ref[...], 0.0)
    n0 = _norm(h0, s0_ref[...], o0_ref[...])
    hop1 = jnp.concatenate([p_ref[0], p_ref[1]], axis=1)
    h1 = jnp.maximum(
        jnp.dot(hop1, w1t_ref[...], preferred_element_type=jnp.float32)
        + b1_ref[...], 0.0)
    n1 = _norm(h1, s1_ref[...], o1_ref[...])
    out_ref[...] = jnp.concatenate([n0, n1], axis=1)


_RB = 400

_dense = pl.pallas_call(
    _dense_body,
    grid=(_N // _RB,),
    in_specs=[
        pl.BlockSpec((_RB, _D), lambda i: (i, 0)),
        pl.BlockSpec((_NC, _RB, _HD), lambda i: (0, i, 0)),
        pl.BlockSpec((_D, _D), lambda i: (0, 0)),
        pl.BlockSpec((_D, _D), lambda i: (0, 0)),
        pl.BlockSpec((1, _D), lambda i: (0, 0)),
        pl.BlockSpec((1, _D), lambda i: (0, 0)),
        pl.BlockSpec((1, _D), lambda i: (0, 0)),
        pl.BlockSpec((1, _D), lambda i: (0, 0)),
        pl.BlockSpec((1, _D), lambda i: (0, 0)),
        pl.BlockSpec((1, _D), lambda i: (0, 0)),
    ],
    out_specs=pl.BlockSpec((_RB, 2 * _D), lambda i: (i, 0)),
    out_shape=jax.ShapeDtypeStruct((_N, 2 * _D), jnp.float32),
)


def kernel(feat_in, edge_index, edge_values, W0, W1, b0, b1,
           offset0, offset1, scale0, scale1):
    pad = _EPAD - _E
    rows3 = jnp.concatenate(
        [edge_index[0], jnp.zeros((pad,), jnp.int32)]).reshape(
            _NS, _NCHUNK, _CH)
    cols3 = jnp.concatenate(
        [edge_index[1], jnp.zeros((pad,), jnp.int32)]).reshape(
            _NS, _NCHUNK, _CH)
    vals3 = jnp.concatenate(
        [edge_values, jnp.zeros((pad,), jnp.float32)]).reshape(
            _NS, _NCHUNK, _CH)
    feat_halves = jnp.stack([feat_in[:, :_HD], feat_in[:, _HD:]])
    part = _get_sc_spmm()(feat_halves, rows3, cols3, vals3)
    return _dense(
        feat_in, part, W0.T, W1.T,
        b0.reshape(1, _D), b1.reshape(1, _D),
        scale0.reshape(1, _D), offset0.reshape(1, _D),
        scale1.reshape(1, _D), offset1.reshape(1, _D),
    )
